# baseline (speedup 1.0000x reference)
import functools

import jax
import jax.numpy as jnp
from jax import lax
from jax.experimental import pallas as pl
from jax.experimental.pallas import tpu as pltpu
from jax.experimental.pallas import tpu_sc as plsc

_LANES = 16


def _sc_body(row_base, rows_per_w, chunk_rows, D, B, n_chunks,
             x_hbm, pos_hbm, out_hbm,
             pos_v0, pos_v1, x_v0, x_v1, o_v0, o_v1,
             sp0, sp1, sx0, sx1, so0, so1):
    pos_v = (pos_v0, pos_v1)
    x_v = (x_v0, x_v1)
    o_v = (o_v0, o_v1)
    sp = (sp0, sp1)
    sx = (sx0, sx1)
    so = (so0, so1)

    wid = lax.axis_index("s") * 2 + lax.axis_index("c")
    row0 = wid * rows_per_w
    in_row0 = row_base + row0
    vec_iters = (chunk_rows * D) // _LANES
    row_iters = D // _LANES
    rsh = row_iters.bit_length() - 1
    jmask = row_iters - 1

    def nxt(c):
        return jnp.where(c + 1 == n_chunks, 0, c + 1)

    def start_pos(c, p):
        pltpu.make_async_copy(
            pos_hbm.at[pl.ds(in_row0 + c * chunk_rows, chunk_rows), :],
            pos_v[p], sp[p]).start()

    def wait_pos(p):
        pltpu.make_async_copy(
            pos_hbm.at[pl.ds(0, chunk_rows), :], pos_v[p], sp[p]).wait()

    def start_x(c, b, p):
        pltpu.make_async_copy(
            x_hbm.at[b, pl.ds(in_row0 + c * chunk_rows, chunk_rows), :],
            x_v[p], sx[p]).start()

    def wait_x(p):
        pltpu.make_async_copy(
            x_hbm.at[0, pl.ds(0, chunk_rows), :], x_v[p], sx[p]).wait()

    def start_out(c, b, p):
        pltpu.make_async_copy(
            o_v[p], out_hbm.at[b, pl.ds(row0 + c * chunk_rows, chunk_rows), :],
            so[p]).start()

    def wait_out(p):
        pltpu.make_async_copy(
            o_v[p], out_hbm.at[0, pl.ds(0, chunk_rows), :], so[p]).wait()

    def item(c, b, par, first):
        xb = b % 2
        wait_x(xb)
        if b == B - 1:
            start_x(nxt(c), 0, 0)
        else:
            start_x(c, b + 1, 1 - xb)
        if not first:
            wait_out(b % 2)

        ob = o_v[b % 2]
        xv = x_v[xb]
        pv = pos_v[par]

        def add_loop(i, _):
            r = i >> rsh
            sl = pl.ds((i & jmask) * _LANES, _LANES)
            ob[r, sl] = xv[r, sl] + pv[r, sl]
            return ()

        lax.fori_loop(0, vec_iters, add_loop, (), unroll=8)
        start_out(c, b, b % 2)

    def do_chunk(c, par, first_pair):
        wait_pos(par)
        start_pos(nxt(c), 1 - par)
        for b in range(B):
            item(c, b, par, first=(first_pair and b < 2))

    start_pos(0, 0)
    start_x(0, 0, 0)
    do_chunk(0, 0, True)
    do_chunk(1, 1, False)

    def pair_body(c2, _):
        do_chunk(2 * c2, 0, False)
        do_chunk(2 * c2 + 1, 1, False)
        return ()

    lax.fori_loop(1, n_chunks // 2, pair_body, ())

    wait_pos(0)
    wait_x(0)
    wait_out(0)
    wait_out(1)


def _sc_add(x, pos, B, D, row_base, sc_rows):
    info = plsc.get_sparse_core_info()
    nw = info.num_cores * info.num_subcores
    rows_per_w = sc_rows // nw
    chunk_rows = next(
        cr for cr in (16, 8)
        if rows_per_w % cr == 0
        and (rows_per_w // cr) >= 2
        and (rows_per_w // cr) % 2 == 0)
    n_chunks = rows_per_w // chunk_rows
    mesh = plsc.VectorSubcoreMesh(core_axis_name="c", subcore_axis_name="s")
    f = pl.kernel(
        functools.partial(_sc_body, row_base, rows_per_w, chunk_rows, D, B,
                          n_chunks),
        mesh=mesh,
        out_type=jax.ShapeDtypeStruct((B, sc_rows, D), jnp.float32),
        scratch_types=(
            [pltpu.VMEM((chunk_rows, D), jnp.float32)] * 6
            + [pltpu.SemaphoreType.DMA] * 6
        ),
    )
    return f(x, pos)


_TC_SEQ_BLOCK = 512


def _tc_body(x_ref, pos_ref, o_ref):
    o_ref[...] = x_ref[...] + pos_ref[...][None, :, :]


def _tc_add(x, pos, B, D, tc_rows):
    bs = _TC_SEQ_BLOCK
    while tc_rows % bs:
        bs //= 2
    return pl.pallas_call(
        _tc_body,
        grid=(tc_rows // bs,),
        in_specs=[
            pl.BlockSpec((B, bs, D), lambda i: (0, i, 0)),
            pl.BlockSpec((bs, D), lambda i: (i, 0)),
        ],
        out_specs=pl.BlockSpec((B, bs, D), lambda i: (0, i, 0)),
        out_shape=jax.ShapeDtypeStruct((B, tc_rows, D), x.dtype),
    )(x, pos)


_SC_ROWS = 3072


def kernel(x, pos_table):
    B, S, D = x.shape
    assert S <= pos_table.shape[0] and S % 32 == 0
    pos = pos_table[:S]
    sc_rows = _SC_ROWS if S > _SC_ROWS else S // 2
    tc_rows = S - sc_rows
    out_sc = _sc_add(x, pos, B, D, tc_rows, sc_rows)
    out_tc = _tc_add(x, pos, B, D, tc_rows)
    return jnp.concatenate([out_tc, out_sc], axis=1)


# speedup vs baseline: 1.1895x; 1.1895x over previous
import functools

import jax
import jax.numpy as jnp
from jax import lax
from jax.experimental import pallas as pl
from jax.experimental.pallas import tpu as pltpu
from jax.experimental.pallas import tpu_sc as plsc

_LANES = 16


def _sc_body(row_base, rows_per_w, chunk_rows, D, B, n_chunks,
             x_hbm, pos_hbm, out_hbm,
             pos_v0, pos_v1, x_v0, x_v1, x_v2, x_v3,
             o_v0, o_v1, o_v2, o_v3,
             sp0, sp1, sx0, sx1, sx2, sx3, so0, so1, so2, so3):
    pos_v = (pos_v0, pos_v1)
    x_v = (x_v0, x_v1, x_v2, x_v3)
    o_v = (o_v0, o_v1, o_v2, o_v3)
    sp = (sp0, sp1)
    sx = (sx0, sx1, sx2, sx3)
    so = (so0, so1, so2, so3)

    wid = lax.axis_index("s") * 2 + lax.axis_index("c")
    row0 = wid * rows_per_w
    in_row0 = row_base + row0
    vec_iters = (chunk_rows * D) // _LANES
    row_iters = D // _LANES
    rsh = row_iters.bit_length() - 1
    jmask = row_iters - 1

    def nxt(c):
        return jnp.where(c + 1 == n_chunks, 0, c + 1)

    def start_pos(c, p):
        pltpu.make_async_copy(
            pos_hbm.at[pl.ds(in_row0 + c * chunk_rows, chunk_rows), :],
            pos_v[p], sp[p]).start()

    def wait_pos(p):
        pltpu.make_async_copy(
            pos_hbm.at[pl.ds(0, chunk_rows), :], pos_v[p], sp[p]).wait()

    def start_x(c, b):
        pltpu.make_async_copy(
            x_hbm.at[b, pl.ds(in_row0 + c * chunk_rows, chunk_rows), :],
            x_v[b], sx[b]).start()

    def wait_x(b):
        pltpu.make_async_copy(
            x_hbm.at[0, pl.ds(0, chunk_rows), :], x_v[b], sx[b]).wait()

    def start_out(c, b):
        pltpu.make_async_copy(
            o_v[b], out_hbm.at[b, pl.ds(row0 + c * chunk_rows, chunk_rows), :],
            so[b]).start()

    def wait_out(b):
        pltpu.make_async_copy(
            o_v[b], out_hbm.at[0, pl.ds(0, chunk_rows), :], so[b]).wait()

    def item(c, b, par, first):
        wait_x(b)
        start_x(nxt(c), b)
        if not first:
            wait_out(b)

        ob = o_v[b]
        xv = x_v[b]
        pv = pos_v[par]

        def add_loop(i, _):
            r = i >> rsh
            sl = pl.ds((i & jmask) * _LANES, _LANES)
            ob[r, sl] = xv[r, sl] + pv[r, sl]
            return ()

        lax.fori_loop(0, vec_iters, add_loop, (), unroll=8)
        start_out(c, b)

    def do_chunk(c, par, first_chunk):
        wait_pos(par)
        start_pos(nxt(c), 1 - par)
        for b in range(B):
            item(c, b, par, first=first_chunk)

    start_pos(0, 0)
    for b in range(B):
        start_x(0, b)
    do_chunk(0, 0, True)
    do_chunk(1, 1, False)

    def pair_body(c2, _):
        do_chunk(2 * c2, 0, False)
        do_chunk(2 * c2 + 1, 1, False)
        return ()

    lax.fori_loop(1, n_chunks // 2, pair_body, ())

    wait_pos(0)
    for b in range(B):
        wait_x(b)
        wait_out(b)


def _sc_add(x, pos, B, D, row_base, sc_rows):
    info = plsc.get_sparse_core_info()
    nw = info.num_cores * info.num_subcores
    rows_per_w = sc_rows // nw
    chunk_rows = next(
        cr for cr in (8, 16)
        if rows_per_w % cr == 0
        and (rows_per_w // cr) >= 2
        and (rows_per_w // cr) % 2 == 0)
    n_chunks = rows_per_w // chunk_rows
    mesh = plsc.VectorSubcoreMesh(core_axis_name="c", subcore_axis_name="s")
    f = pl.kernel(
        functools.partial(_sc_body, row_base, rows_per_w, chunk_rows, D, B,
                          n_chunks),
        mesh=mesh,
        out_type=jax.ShapeDtypeStruct((B, sc_rows, D), jnp.float32),
        scratch_types=(
            [pltpu.VMEM((chunk_rows, D), jnp.float32)] * 10
            + [pltpu.SemaphoreType.DMA] * 10
        ),
    )
    return f(x, pos)


def kernel(x, pos_table):
    B, S, D = x.shape
    assert S <= pos_table.shape[0] and S % 32 == 0
    return _sc_add(x, pos_table[:S], B, D, 0, S)
